# baseline (speedup 1.0000x reference)
import functools

import jax
import jax.numpy as jnp
from jax import lax
from jax.experimental import pallas as pl
from jax.experimental.pallas import tpu as pltpu
from jax.experimental.pallas import tpu_sc as plsc

NC = 2
NS = 16
NW = NC * NS
CHUNK = 256
PIPE = 2
IDXB = 2


def _node_split(n):
  rows = ((n + NS - 1) // NS + 15) // 16 * 16
  last = n - (NS - 1) * rows
  assert last > 0 and last % 16 == 0 and rows % 16 == 0
  return rows, last


def _make_agg(n, d, k_per_tec):
  rows, last = _node_split(n)
  acc_rows = NS * rows + 16
  mesh = plsc.VectorSubcoreMesh(core_axis_name="c", subcore_axis_name="s")

  R = CHUNK
  nf_rows, tail_rows = rows // R, rows % R
  nf_last, tail_last = last // R, last % R

  def body(src_hbm, dst_hbm, hp_hbm, zero_hbm, out_hbm,
           src_v, dst_v, rows_v, acc, sem):
    stage = rows_v[0]
    c = lax.axis_index("c")
    s = lax.axis_index("s")
    wid = c * NS + s
    r0 = s * rows

    def hbm_to_acc(table, nf, tail):
      for j in range(nf):
        o = r0 + j * R
        pltpu.sync_copy(table.at[pl.ds(o, R)], stage)
        pltpu.sync_copy(stage, acc.at[pl.ds(o, R)])
      if tail:
        o = r0 + nf * R
        pltpu.sync_copy(table.at[pl.ds(o, tail)], stage.at[pl.ds(0, tail)])
        pltpu.sync_copy(stage.at[pl.ds(0, tail)], acc.at[pl.ds(o, tail)])

    def init_from(table):
      @pl.when(s < NS - 1)
      def _():
        hbm_to_acc(table, nf_rows, tail_rows)
      @pl.when(s == NS - 1)
      def _():
        hbm_to_acc(table, nf_last, tail_last)

    @pl.when(c == 0)
    def _():
      init_from(hp_hbm)

    @pl.when(c == 1)
    def _():
      init_from(zero_hbm)

    plsc.subcore_barrier()

    kb = k_per_tec // IDXB

    def gather(k, b):
      return pltpu.async_copy(hp_hbm.at[src_v.at[k]], rows_v[b], sem[b])

    for h in range(IDXB):
      pltpu.sync_copy(src_hbm.at[wid, pl.ds(h * kb, kb)], src_v)
      pltpu.sync_copy(dst_hbm.at[wid, pl.ds(h * kb, kb)], dst_v)
      for b in range(PIPE):
        gather(b, b)

      def pair(i, carry):
        for b in range(PIPE):
          k = i * PIPE + b
          pltpu.make_async_copy(hp_hbm.at[src_v.at[k]], rows_v[b],
                                sem[b]).wait()
          pltpu.sync_copy(rows_v[b], acc.at[dst_v.at[k]], add=True)
          nk = k + PIPE

          @pl.when(nk < kb)
          def _():
            gather(nk, b)
        return carry

      lax.fori_loop(0, kb // PIPE, pair, 0)

    plsc.subcore_barrier()

    def acc_to_hbm(nf, tail):
      for j in range(nf):
        o = r0 + j * R
        pltpu.sync_copy(acc.at[pl.ds(o, R)], stage)
        pltpu.sync_copy(stage, out_hbm.at[c, pl.ds(o, R)])
      if tail:
        o = r0 + nf * R
        pltpu.sync_copy(acc.at[pl.ds(o, tail)], stage.at[pl.ds(0, tail)])
        pltpu.sync_copy(stage.at[pl.ds(0, tail)], out_hbm.at[c, pl.ds(o, tail)])

    @pl.when(s < NS - 1)
    def _():
      acc_to_hbm(nf_rows, tail_rows)
    @pl.when(s == NS - 1)
    def _():
      acc_to_hbm(nf_last, tail_last)

  return pl.kernel(
      body,
      out_type=jax.ShapeDtypeStruct((NC, n, d), jnp.float32),
      mesh=mesh,
      compiler_params=pltpu.CompilerParams(use_tc_tiling_on_sc=False),
      scratch_types=[
          pltpu.VMEM((k_per_tec // IDXB, CHUNK), jnp.int32),
          pltpu.VMEM((k_per_tec // IDXB, CHUNK), jnp.int32),
          [pltpu.VMEM((CHUNK, d), jnp.float32)] * PIPE,
          pltpu.VMEM_SHARED((acc_rows, d), jnp.float32),
          [pltpu.SemaphoreType.DMA] * PIPE,
      ],
  )


def _make_deg(n, k_per_tec):
  rows, last = _node_split(n)
  acc_rows = NS * rows + 16
  mesh = plsc.VectorSubcoreMesh(core_axis_name="c", subcore_axis_name="s")

  def body(dst_hbm, out_hbm, dst_v, ones_v, stage_v, acc, sem):
    del sem
    c = lax.axis_index("c")
    s = lax.axis_index("s")
    wid = c * NS + s
    r0 = s * rows

    def zfill(i, carry):
      stage_v[pl.ds(i * 16, 16)] = jnp.zeros((16,), jnp.float32)
      return carry

    lax.fori_loop(0, rows // 16, zfill, 0)

    @pl.when(s < NS - 1)
    def _():
      pltpu.sync_copy(stage_v.at[pl.ds(0, rows)], acc.at[pl.ds(r0, rows)])
    @pl.when(s == NS - 1)
    def _():
      pltpu.sync_copy(stage_v.at[pl.ds(0, last)], acc.at[pl.ds(r0, last)])

    for i in range(CHUNK // 16):
      ones_v[pl.ds(i * 16, 16)] = jnp.ones((16,), jnp.float32)
    pltpu.sync_copy(dst_hbm.at[wid], dst_v)
    plsc.subcore_barrier()

    def step(k, carry):
      pltpu.sync_copy(ones_v, acc.at[dst_v.at[k]], add=True)
      return carry

    lax.fori_loop(0, k_per_tec, step, 0)
    plsc.subcore_barrier()

    def acc_to_hbm(sz):
      pltpu.sync_copy(acc.at[pl.ds(r0, sz)], stage_v.at[pl.ds(0, sz)])
      pltpu.sync_copy(stage_v.at[pl.ds(0, sz)],
                      out_hbm.at[pl.ds(c * n + r0, sz)])

    @pl.when(s < NS - 1)
    def _():
      acc_to_hbm(rows)
    @pl.when(s == NS - 1)
    def _():
      acc_to_hbm(last)

  return pl.kernel(
      body,
      out_type=jax.ShapeDtypeStruct((NC * n,), jnp.float32),
      mesh=mesh,
      compiler_params=pltpu.CompilerParams(use_tc_tiling_on_sc=False),
      scratch_types=[
          pltpu.VMEM((k_per_tec, CHUNK), jnp.int32),
          pltpu.VMEM((CHUNK,), jnp.float32),
          pltpu.VMEM((rows,), jnp.float32),
          pltpu.VMEM_SHARED((acc_rows,), jnp.float32),
          pltpu.SemaphoreType.DMA,
      ],
  )


def _elu(h):
  return jnp.where(h > 0, h, jnp.exp(jnp.minimum(h, 0.0)) - 1.0)


def _mm1_body(degp_ref, x_ref, w_ref, hp_ref, dinv_ref):
  deg = degp_ref[0] + degp_ref[1] + 1.0
  dinv = lax.rsqrt(deg)
  dinv_ref[...] = dinv
  hp_ref[...] = dinv * jnp.dot(x_ref[...], w_ref[...],
                               preferred_element_type=jnp.float32)


def _epimm_body(p_ref, dinv_ref, b_ref, w_ref, o_ref):
  dinv = dinv_ref[...]
  h = _elu(dinv * (p_ref[0] + p_ref[1]) + b_ref[...])
  o_ref[...] = dinv * jnp.dot(h, w_ref[...],
                              preferred_element_type=jnp.float32)


def _epi_body(p_ref, dinv_ref, b_ref, o_ref):
  o_ref[...] = _elu(dinv_ref[...] * (p_ref[0] + p_ref[1]) + b_ref[...])


def kernel(x, edge_index, W1, b1, W2, b2, W3, b3):
  n, d_in = x.shape
  e = edge_index.shape[1]
  d_hid = W1.shape[1]
  d_out = W3.shape[1]
  d3 = 16

  kq = IDXB * PIPE
  k_per_tec = -(-(-(-e // (NW * CHUNK))) // kq) * kq
  e_pad = NW * CHUNK * k_per_tec
  src = edge_index[0].astype(jnp.int32)
  dst = edge_index[1].astype(jnp.int32)
  pad = e_pad - e
  src2d = jnp.concatenate(
      [src, jnp.zeros((pad,), jnp.int32)]).reshape(NW, k_per_tec, CHUNK)
  pad_dst = n + jnp.arange(pad, dtype=jnp.int32) % 128
  dst2d = jnp.concatenate([dst, pad_dst]).reshape(NW, k_per_tec, CHUNK)

  zeros_h = jnp.zeros((n, d_hid), jnp.float32)
  zeros_3 = jnp.zeros((n, d3), jnp.float32)
  W3p = jnp.pad(W3, ((0, 0), (0, d3 - d_out)))
  b3p = jnp.pad(b3, (0, d3 - d_out)).reshape(1, d3)

  agg_h = _make_agg(n, d_hid, k_per_tec)
  agg_3 = _make_agg(n, d3, k_per_tec)
  deg_k = _make_deg(n, k_per_tec)

  bn = 1000
  grid = n // bn
  row_blk = lambda d: pl.BlockSpec((bn, d), lambda i: (i, 0))
  p_blk = lambda d: pl.BlockSpec((NC, bn, d), lambda i: (0, i, 0))
  full = lambda *shape: pl.BlockSpec(shape, lambda i: (0,) * len(shape))

  degp = deg_k(dst2d).reshape(NC, n, 1)

  hp1, dinv = pl.pallas_call(
      _mm1_body,
      grid=(grid,),
      in_specs=[pl.BlockSpec((NC, bn, 1), lambda i: (0, i, 0)),
                row_blk(d_in), full(d_in, d_hid)],
      out_specs=[row_blk(d_hid), row_blk(1)],
      out_shape=[jax.ShapeDtypeStruct((n, d_hid), jnp.float32),
                 jax.ShapeDtypeStruct((n, 1), jnp.float32)],
  )(degp, x, W1)

  p1 = agg_h(src2d, dst2d, hp1, zeros_h)

  hp2 = pl.pallas_call(
      _epimm_body,
      grid=(grid,),
      in_specs=[p_blk(d_hid), row_blk(1), full(1, d_hid),
                full(d_hid, d_hid)],
      out_specs=row_blk(d_hid),
      out_shape=jax.ShapeDtypeStruct((n, d_hid), jnp.float32),
  )(p1, dinv, b1.reshape(1, d_hid), W2)

  p2 = agg_h(src2d, dst2d, hp2, zeros_h)

  hp3 = pl.pallas_call(
      _epimm_body,
      grid=(grid,),
      in_specs=[p_blk(d_hid), row_blk(1), full(1, d_hid),
                full(d_hid, d3)],
      out_specs=row_blk(d3),
      out_shape=jax.ShapeDtypeStruct((n, d3), jnp.float32),
  )(p2, dinv, b2.reshape(1, d_hid), W3p)

  p3 = agg_3(src2d, dst2d, hp3, zeros_3)

  out = pl.pallas_call(
      _epi_body,
      grid=(grid,),
      in_specs=[p_blk(d3), row_blk(1), full(1, d3)],
      out_specs=row_blk(d3),
      out_shape=jax.ShapeDtypeStruct((n, d3), jnp.float32),
  )(p3, dinv, b3p)

  return out[:, :d_out]


# speedup vs baseline: 2.1359x; 2.1359x over previous
import jax
import jax.numpy as jnp
from jax import lax
from jax.experimental import pallas as pl
from jax.experimental.pallas import tpu as pltpu
from jax.experimental.pallas import tpu_sc as plsc

NC = 2
NS = 16
CHUNK = 128
PIPE = 3


def _node_split(n):
  rows = ((n + NS - 1) // NS + 15) // 16 * 16
  last = n - (NS - 1) * rows
  assert last > 0 and last % 16 == 0 and rows % 16 == 0
  return rows, last


def _make_agg(n, dh, k_per_tec):
  rows, last = _node_split(n)
  acc_rows = NS * rows + 16
  mesh = plsc.VectorSubcoreMesh(core_axis_name="c", subcore_axis_name="s")

  R = CHUNK
  nf_rows, tail_rows = rows // R, rows % R
  nf_last, tail_last = last // R, last % R

  def body(src_hbm, dst_hbm, hp_hbm, out_hbm, src_v, dst_v, rows_v,
           tbl, acc, sem):
    stage = rows_v[0]
    c = lax.axis_index("c")
    s = lax.axis_index("s")
    r0 = s * rows

    def load_tbl(nf, tail):
      for j in range(nf):
        o = r0 + j * R
        pltpu.sync_copy(hp_hbm.at[c, pl.ds(o, R)], stage)
        pltpu.sync_copy(stage, tbl.at[pl.ds(o, R)])
        pltpu.sync_copy(stage, acc.at[pl.ds(o, R)])
      if tail:
        o = r0 + nf * R
        st = stage.at[pl.ds(0, tail)]
        pltpu.sync_copy(hp_hbm.at[c, pl.ds(o, tail)], st)
        pltpu.sync_copy(st, tbl.at[pl.ds(o, tail)])
        pltpu.sync_copy(st, acc.at[pl.ds(o, tail)])

    @pl.when(s < NS - 1)
    def _():
      load_tbl(nf_rows, tail_rows)

    @pl.when(s == NS - 1)
    def _():
      load_tbl(nf_last, tail_last)

    pltpu.sync_copy(src_hbm.at[s], src_v)
    pltpu.sync_copy(dst_hbm.at[s], dst_v)
    plsc.subcore_barrier()

    def gather(k, b):
      return pltpu.async_copy(tbl.at[src_v.at[k]], rows_v[b], sem[b])

    for b in range(PIPE):
      gather(b, b)

    def steps(i, carry):
      for b in range(PIPE):
        k = i * PIPE + b

        @pl.when(k < k_per_tec)
        def _():
          pltpu.make_async_copy(tbl.at[src_v.at[k]], rows_v[b],
                                sem[b]).wait()
          pltpu.sync_copy(rows_v[b], acc.at[dst_v.at[k]], add=True)
          nk = k + PIPE

          @pl.when(nk < k_per_tec)
          def _():
            gather(nk, b)
      return carry

    lax.fori_loop(0, -(-k_per_tec // PIPE), steps, 0)
    plsc.subcore_barrier()

    def acc_to_hbm(nf, tail):
      for j in range(nf):
        o = r0 + j * R
        pltpu.sync_copy(acc.at[pl.ds(o, R)], stage)
        pltpu.sync_copy(stage, out_hbm.at[c, pl.ds(o, R)])
      if tail:
        o = r0 + nf * R
        st = stage.at[pl.ds(0, tail)]
        pltpu.sync_copy(acc.at[pl.ds(o, tail)], st)
        pltpu.sync_copy(st, out_hbm.at[c, pl.ds(o, tail)])

    @pl.when(s < NS - 1)
    def _():
      acc_to_hbm(nf_rows, tail_rows)

    @pl.when(s == NS - 1)
    def _():
      acc_to_hbm(nf_last, tail_last)

  return pl.kernel(
      body,
      out_type=jax.ShapeDtypeStruct((NC, n, dh), jnp.float32),
      mesh=mesh,
      compiler_params=pltpu.CompilerParams(use_tc_tiling_on_sc=False),
      scratch_types=[
          pltpu.VMEM((k_per_tec, CHUNK), jnp.int32),
          pltpu.VMEM((k_per_tec, CHUNK), jnp.int32),
          [pltpu.VMEM((CHUNK, dh), jnp.float32)] * PIPE,
          pltpu.VMEM_SHARED((n, dh), jnp.float32),
          pltpu.VMEM_SHARED((acc_rows, dh), jnp.float32),
          [pltpu.SemaphoreType.DMA] * PIPE,
      ],
  )


def _make_deg(n, k_per_tec):
  rows, last = _node_split(n)
  acc_rows = NS * rows + 16
  kh = k_per_tec // 2
  mesh = plsc.VectorSubcoreMesh(core_axis_name="c", subcore_axis_name="s")

  def body(dst_hbm, out_hbm, dst_v, ones_v, stage_v, acc):
    c = lax.axis_index("c")
    s = lax.axis_index("s")
    r0 = s * rows

    def zfill(i, carry):
      stage_v[pl.ds(i * 16, 16)] = jnp.zeros((16,), jnp.float32)
      return carry

    lax.fori_loop(0, rows // 16, zfill, 0)

    @pl.when(s < NS - 1)
    def _():
      pltpu.sync_copy(stage_v.at[pl.ds(0, rows)], acc.at[pl.ds(r0, rows)])

    @pl.when(s == NS - 1)
    def _():
      pltpu.sync_copy(stage_v.at[pl.ds(0, last)], acc.at[pl.ds(r0, last)])

    for i in range(CHUNK // 16):
      ones_v[pl.ds(i * 16, 16)] = jnp.ones((16,), jnp.float32)
    pltpu.sync_copy(dst_hbm.at[s, pl.ds(c * kh, kh)], dst_v)
    plsc.subcore_barrier()

    def step(k, carry):
      pltpu.sync_copy(ones_v, acc.at[dst_v.at[k]], add=True)
      return carry

    lax.fori_loop(0, kh, step, 0)
    plsc.subcore_barrier()

    def acc_to_hbm(sz):
      pltpu.sync_copy(acc.at[pl.ds(r0, sz)], stage_v.at[pl.ds(0, sz)])
      pltpu.sync_copy(stage_v.at[pl.ds(0, sz)],
                      out_hbm.at[pl.ds(c * n + r0, sz)])

    @pl.when(s < NS - 1)
    def _():
      acc_to_hbm(rows)

    @pl.when(s == NS - 1)
    def _():
      acc_to_hbm(last)

  return pl.kernel(
      body,
      out_type=jax.ShapeDtypeStruct((NC * n,), jnp.float32),
      mesh=mesh,
      compiler_params=pltpu.CompilerParams(use_tc_tiling_on_sc=False),
      scratch_types=[
          pltpu.VMEM((kh, CHUNK), jnp.int32),
          pltpu.VMEM((CHUNK,), jnp.float32),
          pltpu.VMEM((rows,), jnp.float32),
          pltpu.VMEM_SHARED((acc_rows,), jnp.float32),
      ],
  )


def _elu(h):
  return jnp.where(h > 0, h, jnp.exp(jnp.minimum(h, 0.0)) - 1.0)


def _mm1_body(degp_ref, x_ref, w_ref, hp_ref, dinv_ref):
  deg = degp_ref[0] + degp_ref[1] + 1.0
  dinv = lax.rsqrt(deg)
  dinv_ref[...] = dinv
  t = dinv * jnp.dot(x_ref[...], w_ref[...],
                     preferred_element_type=jnp.float32)
  dh = t.shape[1] // 2
  hp_ref[0] = t[:, :dh]
  hp_ref[1] = t[:, dh:]


def _epimm_body(p_ref, dinv_ref, b_ref, w_ref, o_ref):
  dinv = dinv_ref[...]
  h = _elu(jnp.concatenate([p_ref[0], p_ref[1]], axis=1) * dinv + b_ref[...])
  t = dinv * jnp.dot(h, w_ref[...], preferred_element_type=jnp.float32)
  dh = t.shape[1] // 2
  o_ref[0] = t[:, :dh]
  o_ref[1] = t[:, dh:]


def _epi_body(p_ref, dinv_ref, b_ref, o_ref):
  h = jnp.concatenate([p_ref[0], p_ref[1]], axis=1)
  o_ref[...] = _elu(h * dinv_ref[...] + b_ref[...])


def kernel(x, edge_index, W1, b1, W2, b2, W3, b3):
  n, d_in = x.shape
  e = edge_index.shape[1]
  d_hid = W1.shape[1]
  d_out = W3.shape[1]
  dh = d_hid // 2
  d3 = 32
  dh3 = d3 // 2

  k_per_tec = -(-(-(-e // (NS * CHUNK))) // 2) * 2
  e_pad = NS * CHUNK * k_per_tec
  src = edge_index[0].astype(jnp.int32)
  dst = edge_index[1].astype(jnp.int32)
  pad = e_pad - e
  pad_dst = n + jnp.arange(pad, dtype=jnp.int32) % 128
  src2d = jnp.concatenate(
      [src, jnp.zeros((pad,), jnp.int32)]).reshape(NS, k_per_tec, CHUNK)
  dst2d = jnp.concatenate([dst, pad_dst]).reshape(NS, k_per_tec, CHUNK)

  W3p = jnp.pad(W3, ((0, 0), (0, d3 - d_out)))
  b3p = jnp.pad(b3, (0, d3 - d_out)).reshape(1, d3)

  agg_h = _make_agg(n, dh, k_per_tec)
  agg_3 = _make_agg(n, dh3, k_per_tec)
  deg_k = _make_deg(n, k_per_tec)

  bn = 1000
  grid = n // bn
  row_blk = lambda d: pl.BlockSpec((bn, d), lambda i: (i, 0))
  p_blk = lambda d: pl.BlockSpec((NC, bn, d), lambda i: (0, i, 0))
  full = lambda *shape: pl.BlockSpec(shape, lambda i: (0,) * len(shape))

  degp = deg_k(dst2d).reshape(NC, n, 1)

  hp1, dinv = pl.pallas_call(
      _mm1_body,
      grid=(grid,),
      in_specs=[pl.BlockSpec((NC, bn, 1), lambda i: (0, i, 0)),
                row_blk(d_in), full(d_in, d_hid)],
      out_specs=[p_blk(dh), row_blk(1)],
      out_shape=[jax.ShapeDtypeStruct((NC, n, dh), jnp.float32),
                 jax.ShapeDtypeStruct((n, 1), jnp.float32)],
  )(degp, x, W1)

  p1 = agg_h(src2d, dst2d, hp1)

  hp2 = pl.pallas_call(
      _epimm_body,
      grid=(grid,),
      in_specs=[p_blk(dh), row_blk(1), full(1, d_hid), full(d_hid, d_hid)],
      out_specs=p_blk(dh),
      out_shape=jax.ShapeDtypeStruct((NC, n, dh), jnp.float32),
  )(p1, dinv, b1.reshape(1, d_hid), W2)

  p2 = agg_h(src2d, dst2d, hp2)

  hp3 = pl.pallas_call(
      _epimm_body,
      grid=(grid,),
      in_specs=[p_blk(dh), row_blk(1), full(1, d_hid), full(d_hid, d3)],
      out_specs=p_blk(dh3),
      out_shape=jax.ShapeDtypeStruct((NC, n, dh3), jnp.float32),
  )(p2, dinv, b2.reshape(1, d_hid), W3p)

  p3 = agg_3(src2d, dst2d, hp3)

  out = pl.pallas_call(
      _epi_body,
      grid=(grid,),
      in_specs=[p_blk(dh3), row_blk(1), full(1, d3)],
      out_specs=row_blk(d3),
      out_shape=jax.ShapeDtypeStruct((n, d3), jnp.float32),
  )(p3, dinv, b3p)

  return out[:, :d_out]


# speedup vs baseline: 2.1733x; 1.0175x over previous
import jax
import jax.numpy as jnp
from jax import lax
from jax.experimental import pallas as pl
from jax.experimental.pallas import tpu as pltpu
from jax.experimental.pallas import tpu_sc as plsc

NC = 2
NS = 16
CHUNK = 128
PIPE = 4


def _node_split(n):
  rows = ((n + NS - 1) // NS + 15) // 16 * 16
  last = n - (NS - 1) * rows
  assert last > 0 and last % 16 == 0 and rows % 16 == 0
  return rows, last


def _make_agg(n, dh, k_per_tec):
  rows, last = _node_split(n)
  acc_rows = NS * rows + 16
  mesh = plsc.VectorSubcoreMesh(core_axis_name="c", subcore_axis_name="s")

  R = CHUNK
  nf_rows, tail_rows = rows // R, rows % R
  nf_last, tail_last = last // R, last % R

  def body(src_hbm, dst_hbm, hp_hbm, out_hbm, src_v, dst_v, rows_v,
           tbl, acc, sem):
    stage = rows_v[0]
    c = lax.axis_index("c")
    s = lax.axis_index("s")
    r0 = s * rows

    def load_tbl(nf, tail):
      hops = [(r0 + j * R, R) for j in range(nf)]
      if tail:
        hops.append((r0 + nf * R, tail))
      for j, (o, sz) in enumerate(hops[:PIPE]):
        pltpu.async_copy(hp_hbm.at[c, pl.ds(o, sz)],
                         rows_v[j].at[pl.ds(0, sz)], sem[j])
      for j, (o, sz) in enumerate(hops):
        st = rows_v[j % PIPE].at[pl.ds(0, sz)]
        pltpu.make_async_copy(hp_hbm.at[c, pl.ds(o, sz)], st,
                              sem[j % PIPE]).wait()
        pltpu.sync_copy(st, tbl.at[pl.ds(o, sz)])
        pltpu.sync_copy(st, acc.at[pl.ds(o, sz)])
        nj = j + PIPE
        if nj < len(hops):
          no, nsz = hops[nj]
          pltpu.async_copy(hp_hbm.at[c, pl.ds(no, nsz)],
                           rows_v[j % PIPE].at[pl.ds(0, nsz)], sem[j % PIPE])

    @pl.when(s < NS - 1)
    def _():
      load_tbl(nf_rows, tail_rows)

    @pl.when(s == NS - 1)
    def _():
      load_tbl(nf_last, tail_last)

    pltpu.sync_copy(src_hbm.at[s], src_v)
    pltpu.sync_copy(dst_hbm.at[s], dst_v)
    plsc.subcore_barrier()

    def gather(k, b):
      return pltpu.async_copy(tbl.at[src_v.at[k]], rows_v[b], sem[b])

    for b in range(PIPE):
      gather(b, b)

    def steps(i, carry):
      for b in range(PIPE):
        k = i * PIPE + b

        @pl.when(k < k_per_tec)
        def _():
          pltpu.make_async_copy(tbl.at[src_v.at[k]], rows_v[b],
                                sem[b]).wait()
          pltpu.sync_copy(rows_v[b], acc.at[dst_v.at[k]], add=True)
          nk = k + PIPE

          @pl.when(nk < k_per_tec)
          def _():
            gather(nk, b)
      return carry

    lax.fori_loop(0, -(-k_per_tec // PIPE), steps, 0)
    plsc.subcore_barrier()

    def acc_to_hbm(nf, tail):
      for j in range(nf):
        o = r0 + j * R
        pltpu.sync_copy(acc.at[pl.ds(o, R)], stage)
        pltpu.sync_copy(stage, out_hbm.at[c, pl.ds(o, R)])
      if tail:
        o = r0 + nf * R
        st = stage.at[pl.ds(0, tail)]
        pltpu.sync_copy(acc.at[pl.ds(o, tail)], st)
        pltpu.sync_copy(st, out_hbm.at[c, pl.ds(o, tail)])

    @pl.when(s < NS - 1)
    def _():
      acc_to_hbm(nf_rows, tail_rows)

    @pl.when(s == NS - 1)
    def _():
      acc_to_hbm(nf_last, tail_last)

  return pl.kernel(
      body,
      out_type=jax.ShapeDtypeStruct((NC, n, dh), jnp.float32),
      mesh=mesh,
      compiler_params=pltpu.CompilerParams(use_tc_tiling_on_sc=False),
      scratch_types=[
          pltpu.VMEM((k_per_tec, CHUNK), jnp.int32),
          pltpu.VMEM((k_per_tec, CHUNK), jnp.int32),
          [pltpu.VMEM((CHUNK, dh), jnp.float32)] * PIPE,
          pltpu.VMEM_SHARED((n, dh), jnp.float32),
          pltpu.VMEM_SHARED((acc_rows, dh), jnp.float32),
          [pltpu.SemaphoreType.DMA] * PIPE,
      ],
  )


def _make_deg(n, k_per_tec):
  rows, last = _node_split(n)
  acc_rows = NS * rows + 16
  kh = k_per_tec // 2
  mesh = plsc.VectorSubcoreMesh(core_axis_name="c", subcore_axis_name="s")

  def body(dst_hbm, out_hbm, dst_v, ones_v, stage_v, acc):
    c = lax.axis_index("c")
    s = lax.axis_index("s")
    r0 = s * rows

    def zfill(i, carry):
      stage_v[pl.ds(i * 16, 16)] = jnp.zeros((16,), jnp.float32)
      return carry

    lax.fori_loop(0, rows // 16, zfill, 0)

    @pl.when(s < NS - 1)
    def _():
      pltpu.sync_copy(stage_v.at[pl.ds(0, rows)], acc.at[pl.ds(r0, rows)])

    @pl.when(s == NS - 1)
    def _():
      pltpu.sync_copy(stage_v.at[pl.ds(0, last)], acc.at[pl.ds(r0, last)])

    for i in range(CHUNK // 16):
      ones_v[pl.ds(i * 16, 16)] = jnp.ones((16,), jnp.float32)
    pltpu.sync_copy(dst_hbm.at[s, pl.ds(c * kh, kh)], dst_v)
    plsc.subcore_barrier()

    def step(k, carry):
      pltpu.sync_copy(ones_v, acc.at[dst_v.at[k]], add=True)
      return carry

    lax.fori_loop(0, kh, step, 0)
    plsc.subcore_barrier()

    def acc_to_hbm(sz):
      pltpu.sync_copy(acc.at[pl.ds(r0, sz)], stage_v.at[pl.ds(0, sz)])
      pltpu.sync_copy(stage_v.at[pl.ds(0, sz)],
                      out_hbm.at[pl.ds(c * n + r0, sz)])

    @pl.when(s < NS - 1)
    def _():
      acc_to_hbm(rows)

    @pl.when(s == NS - 1)
    def _():
      acc_to_hbm(last)

  return pl.kernel(
      body,
      out_type=jax.ShapeDtypeStruct((NC * n,), jnp.float32),
      mesh=mesh,
      compiler_params=pltpu.CompilerParams(use_tc_tiling_on_sc=False),
      scratch_types=[
          pltpu.VMEM((kh, CHUNK), jnp.int32),
          pltpu.VMEM((CHUNK,), jnp.float32),
          pltpu.VMEM((rows,), jnp.float32),
          pltpu.VMEM_SHARED((acc_rows,), jnp.float32),
      ],
  )


def _elu(h):
  return jnp.where(h > 0, h, jnp.exp(jnp.minimum(h, 0.0)) - 1.0)


def _mm1_body(degp_ref, x_ref, w_ref, hp_ref, dinv_ref):
  deg = degp_ref[0] + degp_ref[1] + 1.0
  dinv = lax.rsqrt(deg)
  dinv_ref[...] = dinv
  t = dinv * jnp.dot(x_ref[...], w_ref[...],
                     preferred_element_type=jnp.float32)
  dh = t.shape[1] // 2
  hp_ref[0] = t[:, :dh]
  hp_ref[1] = t[:, dh:]


def _epimm_body(p_ref, dinv_ref, b_ref, w_ref, o_ref):
  dinv = dinv_ref[...]
  h = _elu(jnp.concatenate([p_ref[0], p_ref[1]], axis=1) * dinv + b_ref[...])
  t = dinv * jnp.dot(h, w_ref[...], preferred_element_type=jnp.float32)
  dh = t.shape[1] // 2
  o_ref[0] = t[:, :dh]
  o_ref[1] = t[:, dh:]


def _epi_body(p_ref, dinv_ref, b_ref, o_ref):
  h = jnp.concatenate([p_ref[0], p_ref[1]], axis=1)
  d_out = o_ref.shape[1]
  o_ref[...] = _elu(h * dinv_ref[...] + b_ref[...])[:, :d_out]


def kernel(x, edge_index, W1, b1, W2, b2, W3, b3):
  n, d_in = x.shape
  e = edge_index.shape[1]
  d_hid = W1.shape[1]
  d_out = W3.shape[1]
  dh = d_hid // 2
  d3 = 32
  dh3 = d3 // 2

  k_per_tec = -(-(-(-e // (NS * CHUNK))) // 2) * 2
  e_pad = NS * CHUNK * k_per_tec
  src = edge_index[0].astype(jnp.int32)
  dst = edge_index[1].astype(jnp.int32)
  pad = e_pad - e
  pad_dst = n + jnp.arange(pad, dtype=jnp.int32) % 128
  src2d = jnp.concatenate(
      [src, jnp.zeros((pad,), jnp.int32)]).reshape(NS, k_per_tec, CHUNK)
  dst2d = jnp.concatenate([dst, pad_dst]).reshape(NS, k_per_tec, CHUNK)

  W3p = jnp.pad(W3, ((0, 0), (0, d3 - d_out)))
  b3p = jnp.pad(b3, (0, d3 - d_out)).reshape(1, d3)

  agg_h = _make_agg(n, dh, k_per_tec)
  agg_3 = _make_agg(n, dh3, k_per_tec)
  deg_k = _make_deg(n, k_per_tec)

  bn = 1000
  grid = n // bn
  row_blk = lambda d: pl.BlockSpec((bn, d), lambda i: (i, 0))
  p_blk = lambda d: pl.BlockSpec((NC, bn, d), lambda i: (0, i, 0))
  full = lambda *shape: pl.BlockSpec(shape, lambda i: (0,) * len(shape))

  degp = deg_k(dst2d).reshape(NC, n, 1)

  hp1, dinv = pl.pallas_call(
      _mm1_body,
      grid=(grid,),
      in_specs=[pl.BlockSpec((NC, bn, 1), lambda i: (0, i, 0)),
                row_blk(d_in), full(d_in, d_hid)],
      out_specs=[p_blk(dh), row_blk(1)],
      out_shape=[jax.ShapeDtypeStruct((NC, n, dh), jnp.float32),
                 jax.ShapeDtypeStruct((n, 1), jnp.float32)],
  )(degp, x, W1)

  p1 = agg_h(src2d, dst2d, hp1)

  hp2 = pl.pallas_call(
      _epimm_body,
      grid=(grid,),
      in_specs=[p_blk(dh), row_blk(1), full(1, d_hid), full(d_hid, d_hid)],
      out_specs=p_blk(dh),
      out_shape=jax.ShapeDtypeStruct((NC, n, dh), jnp.float32),
  )(p1, dinv, b1.reshape(1, d_hid), W2)

  p2 = agg_h(src2d, dst2d, hp2)

  hp3 = pl.pallas_call(
      _epimm_body,
      grid=(grid,),
      in_specs=[p_blk(dh), row_blk(1), full(1, d_hid), full(d_hid, d3)],
      out_specs=p_blk(dh3),
      out_shape=jax.ShapeDtypeStruct((NC, n, dh3), jnp.float32),
  )(p2, dinv, b2.reshape(1, d_hid), W3p)

  p3 = agg_3(src2d, dst2d, hp3)

  out = pl.pallas_call(
      _epi_body,
      grid=(grid,),
      in_specs=[p_blk(dh3), row_blk(1), full(1, d3)],
      out_specs=row_blk(d_out),
      out_shape=jax.ShapeDtypeStruct((n, d_out), jnp.float32),
  )(p3, dinv, b3p)

  return out


# speedup vs baseline: 2.2502x; 1.0354x over previous
import jax
import jax.numpy as jnp
from jax import lax
from jax.experimental import pallas as pl
from jax.experimental.pallas import tpu as pltpu
from jax.experimental.pallas import tpu_sc as plsc

NC = 2
NS = 16
CHUNK = 128
PIPE = 4


def _node_split(n):
  rows = ((n + NS - 1) // NS + 15) // 16 * 16
  last = n - (NS - 1) * rows
  assert last > 0 and last % 16 == 0 and rows % 16 == 0
  return rows, last


def _make_agg(n, dh, k_per_tec):
  rows, last = _node_split(n)
  acc_rows = NS * rows + 16
  mesh = plsc.VectorSubcoreMesh(core_axis_name="c", subcore_axis_name="s")

  R = CHUNK
  nf_rows, tail_rows = rows // R, rows % R
  nf_last, tail_last = last // R, last % R

  def body(src_hbm, dst_hbm, hp_hbm, out_hbm, src_v, dst_v, rows_v,
           tbl, acc, sem):
    stage = rows_v[0]
    c = lax.axis_index("c")
    s = lax.axis_index("s")
    r0 = s * rows

    def load_tbl(nf, tail):
      hops = [(r0 + j * R, R) for j in range(nf)]
      if tail:
        hops.append((r0 + nf * R, tail))
      for j, (o, sz) in enumerate(hops[:PIPE]):
        pltpu.async_copy(hp_hbm.at[c, pl.ds(o, sz)],
                         rows_v[j].at[pl.ds(0, sz)], sem[j])
      for j, (o, sz) in enumerate(hops):
        st = rows_v[j % PIPE].at[pl.ds(0, sz)]
        pltpu.make_async_copy(hp_hbm.at[c, pl.ds(o, sz)], st,
                              sem[j % PIPE]).wait()
        pltpu.sync_copy(st, tbl.at[pl.ds(o, sz)])
        pltpu.sync_copy(st, acc.at[pl.ds(o, sz)])
        nj = j + PIPE
        if nj < len(hops):
          no, nsz = hops[nj]
          pltpu.async_copy(hp_hbm.at[c, pl.ds(no, nsz)],
                           rows_v[j % PIPE].at[pl.ds(0, nsz)], sem[j % PIPE])

    @pl.when(s < NS - 1)
    def _():
      load_tbl(nf_rows, tail_rows)

    @pl.when(s == NS - 1)
    def _():
      load_tbl(nf_last, tail_last)

    pltpu.sync_copy(src_hbm.at[s], src_v)
    pltpu.sync_copy(dst_hbm.at[s], dst_v)
    plsc.subcore_barrier()

    def gather(k, b):
      return pltpu.async_copy(tbl.at[src_v.at[k]], rows_v[b], sem[b])

    for b in range(PIPE):
      gather(b, b)

    def steps(i, carry):
      for b in range(PIPE):
        k = i * PIPE + b

        @pl.when(k < k_per_tec)
        def _():
          pltpu.make_async_copy(tbl.at[src_v.at[k]], rows_v[b],
                                sem[b]).wait()
          pltpu.sync_copy(rows_v[b], acc.at[dst_v.at[k]], add=True)
          nk = k + PIPE

          @pl.when(nk < k_per_tec)
          def _():
            gather(nk, b)
      return carry

    lax.fori_loop(0, -(-k_per_tec // PIPE), steps, 0)
    plsc.subcore_barrier()

    def acc_to_hbm(nf, tail):
      for j in range(nf):
        o = r0 + j * R
        pltpu.sync_copy(acc.at[pl.ds(o, R)], stage)
        pltpu.sync_copy(stage, out_hbm.at[c, pl.ds(o, R)])
      if tail:
        o = r0 + nf * R
        st = stage.at[pl.ds(0, tail)]
        pltpu.sync_copy(acc.at[pl.ds(o, tail)], st)
        pltpu.sync_copy(st, out_hbm.at[c, pl.ds(o, tail)])

    @pl.when(s < NS - 1)
    def _():
      acc_to_hbm(nf_rows, tail_rows)

    @pl.when(s == NS - 1)
    def _():
      acc_to_hbm(nf_last, tail_last)

  return pl.kernel(
      body,
      out_type=jax.ShapeDtypeStruct((NC, n, dh), jnp.float32),
      mesh=mesh,
      compiler_params=pltpu.CompilerParams(use_tc_tiling_on_sc=False),
      scratch_types=[
          pltpu.VMEM((k_per_tec, CHUNK), jnp.int32),
          pltpu.VMEM((k_per_tec, CHUNK), jnp.int32),
          [pltpu.VMEM((CHUNK, dh), jnp.float32)] * PIPE,
          pltpu.VMEM_SHARED((n, dh), jnp.float32),
          pltpu.VMEM_SHARED((acc_rows, dh), jnp.float32),
          [pltpu.SemaphoreType.DMA] * PIPE,
      ],
  )


def _make_deg(n, k_per_tec):
  rows, last = _node_split(n)
  acc_rows = NS * rows + 16
  kh = k_per_tec // 2
  mesh = plsc.VectorSubcoreMesh(core_axis_name="c", subcore_axis_name="s")

  def body(dst_hbm, out0_hbm, out1_hbm, dst_v, ones_v, stage_v, acc):
    c = lax.axis_index("c")
    s = lax.axis_index("s")
    r0 = s * rows

    def zfill(i, carry):
      stage_v[pl.ds(i * 16, 16)] = jnp.zeros((16,), jnp.float32)
      return carry

    lax.fori_loop(0, rows // 16, zfill, 0)

    @pl.when(s < NS - 1)
    def _():
      pltpu.sync_copy(stage_v.at[pl.ds(0, rows)], acc.at[pl.ds(r0, rows)])

    @pl.when(s == NS - 1)
    def _():
      pltpu.sync_copy(stage_v.at[pl.ds(0, last)], acc.at[pl.ds(r0, last)])

    for i in range(CHUNK // 16):
      ones_v[pl.ds(i * 16, 16)] = jnp.ones((16,), jnp.float32)
    pltpu.sync_copy(dst_hbm.at[s, pl.ds(c * kh, kh)], dst_v)
    plsc.subcore_barrier()

    def step(k, carry):
      pltpu.sync_copy(ones_v, acc.at[dst_v.at[k]], add=True)
      return carry

    lax.fori_loop(0, kh, step, 0)
    plsc.subcore_barrier()

    def acc_to_hbm(sz):
      pltpu.sync_copy(acc.at[pl.ds(r0, sz)], stage_v.at[pl.ds(0, sz)])

      @pl.when(c == 0)
      def _():
        pltpu.sync_copy(stage_v.at[pl.ds(0, sz)], out0_hbm.at[pl.ds(r0, sz)])

      @pl.when(c == 1)
      def _():
        pltpu.sync_copy(stage_v.at[pl.ds(0, sz)], out1_hbm.at[pl.ds(r0, sz)])

    @pl.when(s < NS - 1)
    def _():
      acc_to_hbm(rows)

    @pl.when(s == NS - 1)
    def _():
      acc_to_hbm(last)

  return pl.kernel(
      body,
      out_type=[jax.ShapeDtypeStruct((n,), jnp.float32),
                jax.ShapeDtypeStruct((n,), jnp.float32)],
      mesh=mesh,
      compiler_params=pltpu.CompilerParams(use_tc_tiling_on_sc=False),
      scratch_types=[
          pltpu.VMEM((kh, CHUNK), jnp.int32),
          pltpu.VMEM((CHUNK,), jnp.float32),
          pltpu.VMEM((rows,), jnp.float32),
          pltpu.VMEM_SHARED((acc_rows,), jnp.float32),
      ],
  )


def _elu(h):
  return jnp.where(h > 0, h, jnp.exp(jnp.minimum(h, 0.0)) - 1.0)


def _mm1_body(d0_ref, d1_ref, x_ref, w_ref, hp_ref, dinv_ref):
  i = pl.program_id(0)
  deg = d0_ref[i] + d1_ref[i] + 1.0
  dinv = lax.rsqrt(deg)
  dinv_ref[i] = dinv
  t = dinv[:, None] * jnp.dot(x_ref[...], w_ref[...],
                              preferred_element_type=jnp.float32)
  dh = t.shape[1] // 2
  hp_ref[0] = t[:, :dh]
  hp_ref[1] = t[:, dh:]


def _epimm_body(p_ref, dinv_ref, b_ref, w_ref, o_ref):
  dinv = dinv_ref[pl.program_id(0)][:, None]
  h = _elu(jnp.concatenate([p_ref[0], p_ref[1]], axis=1) * dinv + b_ref[...])
  t = dinv * jnp.dot(h, w_ref[...], preferred_element_type=jnp.float32)
  dh = t.shape[1] // 2
  o_ref[0] = t[:, :dh]
  o_ref[1] = t[:, dh:]


def _epi_body(p_ref, dinv_ref, b_ref, o_ref):
  h = jnp.concatenate([p_ref[0], p_ref[1]], axis=1)
  d_out = o_ref.shape[1]
  o_ref[...] = _elu(h * dinv_ref[pl.program_id(0)][:, None] + b_ref[...])[:, :d_out]


def kernel(x, edge_index, W1, b1, W2, b2, W3, b3):
  n, d_in = x.shape
  e = edge_index.shape[1]
  d_hid = W1.shape[1]
  d_out = W3.shape[1]
  dh = d_hid // 2
  d3 = 32
  dh3 = d3 // 2

  k_per_tec = -(-(-(-e // (NS * CHUNK))) // 2) * 2
  e_pad = NS * CHUNK * k_per_tec
  src = edge_index[0].astype(jnp.int32)
  dst = edge_index[1].astype(jnp.int32)
  pad = e_pad - e
  pad_dst = n + jnp.arange(pad, dtype=jnp.int32) % 128
  src2d = jnp.concatenate(
      [src, jnp.zeros((pad,), jnp.int32)]).reshape(NS, k_per_tec, CHUNK)
  dst2d = jnp.concatenate([dst, pad_dst]).reshape(NS, k_per_tec, CHUNK)

  W3p = jnp.pad(W3, ((0, 0), (0, d3 - d_out)))
  b3p = jnp.pad(b3, (0, d3 - d_out)).reshape(1, d3)

  agg_h = _make_agg(n, dh, k_per_tec)
  agg_3 = _make_agg(n, dh3, k_per_tec)
  deg_k = _make_deg(n, k_per_tec)

  bn = 1000
  grid = n // bn
  row_blk = lambda d: pl.BlockSpec((bn, d), lambda i: (i, 0))
  v_blk = pl.BlockSpec((grid, bn), lambda i: (0, 0))
  p_blk = lambda d: pl.BlockSpec((NC, bn, d), lambda i: (0, i, 0))
  full = lambda *shape: pl.BlockSpec(shape, lambda i: (0,) * len(shape))

  deg0, deg1 = deg_k(dst2d)

  hp1, dinv = pl.pallas_call(
      _mm1_body,
      grid=(grid,),
      in_specs=[v_blk, v_blk, row_blk(d_in), full(d_in, d_hid)],
      out_specs=[p_blk(dh), v_blk],
      out_shape=[jax.ShapeDtypeStruct((NC, n, dh), jnp.float32),
                 jax.ShapeDtypeStruct((grid, bn), jnp.float32)],
  )(deg0.reshape(grid, bn), deg1.reshape(grid, bn), x, W1)

  p1 = agg_h(src2d, dst2d, hp1)

  hp2 = pl.pallas_call(
      _epimm_body,
      grid=(grid,),
      in_specs=[p_blk(dh), v_blk, full(1, d_hid), full(d_hid, d_hid)],
      out_specs=p_blk(dh),
      out_shape=jax.ShapeDtypeStruct((NC, n, dh), jnp.float32),
  )(p1, dinv, b1.reshape(1, d_hid), W2)

  p2 = agg_h(src2d, dst2d, hp2)

  hp3 = pl.pallas_call(
      _epimm_body,
      grid=(grid,),
      in_specs=[p_blk(dh), v_blk, full(1, d_hid), full(d_hid, d3)],
      out_specs=p_blk(dh3),
      out_shape=jax.ShapeDtypeStruct((NC, n, dh3), jnp.float32),
  )(p2, dinv, b2.reshape(1, d_hid), W3p)

  p3 = agg_3(src2d, dst2d, hp3)

  out = pl.pallas_call(
      _epi_body,
      grid=(grid,),
      in_specs=[p_blk(dh3), v_blk, full(1, d3)],
      out_specs=row_blk(d_out),
      out_shape=jax.ShapeDtypeStruct((n, d_out), jnp.float32),
  )(p3, dinv, b3p)

  return out


# speedup vs baseline: 2.2921x; 1.0186x over previous
import jax
import jax.numpy as jnp
from jax import lax
from jax.experimental import pallas as pl
from jax.experimental.pallas import tpu as pltpu
from jax.experimental.pallas import tpu_sc as plsc

NC = 2
NS = 16
CHUNK = 128
PIPE = 4


def _node_split(n):
  rows = ((n + NS - 1) // NS + 15) // 16 * 16
  last = n - (NS - 1) * rows
  assert last > 0 and last % 16 == 0 and rows % 16 == 0
  return rows, last


def _make_agg(n, dh, k_per_tec):
  rows, last = _node_split(n)
  acc_rows = NS * rows + 16
  mesh = plsc.VectorSubcoreMesh(core_axis_name="c", subcore_axis_name="s")

  R = CHUNK
  nf_rows, tail_rows = rows // R, rows % R
  nf_last, tail_last = last // R, last % R

  def body(src_hbm, dst_hbm, hp_hbm, out_hbm, src_v, dst_v, rows_v,
           tbl, acc, sem):
    stage = rows_v[0]
    c = lax.axis_index("c")
    s = lax.axis_index("s")
    r0 = s * rows

    def load_tbl(nf, tail):
      hops = [(r0 + j * R, R) for j in range(nf)]
      if tail:
        hops.append((r0 + nf * R, tail))
      for j, (o, sz) in enumerate(hops[:PIPE]):
        pltpu.async_copy(hp_hbm.at[c, pl.ds(o, sz)],
                         rows_v[j].at[pl.ds(0, sz)], sem[j])
      for j, (o, sz) in enumerate(hops):
        st = rows_v[j % PIPE].at[pl.ds(0, sz)]
        pltpu.make_async_copy(hp_hbm.at[c, pl.ds(o, sz)], st,
                              sem[j % PIPE]).wait()
        pltpu.sync_copy(st, tbl.at[pl.ds(o, sz)])
        pltpu.sync_copy(st, acc.at[pl.ds(o, sz)])
        nj = j + PIPE
        if nj < len(hops):
          no, nsz = hops[nj]
          pltpu.async_copy(hp_hbm.at[c, pl.ds(no, nsz)],
                           rows_v[j % PIPE].at[pl.ds(0, nsz)], sem[j % PIPE])

    @pl.when(s < NS - 1)
    def _():
      load_tbl(nf_rows, tail_rows)

    @pl.when(s == NS - 1)
    def _():
      load_tbl(nf_last, tail_last)

    pltpu.sync_copy(src_hbm.at[s], src_v)
    pltpu.sync_copy(dst_hbm.at[s], dst_v)
    plsc.subcore_barrier()

    def gather(k, b):
      return pltpu.async_copy(tbl.at[src_v.at[k]], rows_v[b], sem[b])

    for b in range(PIPE):
      gather(b, b)

    def steps(i, carry):
      for b in range(PIPE):
        k = i * PIPE + b

        @pl.when(k < k_per_tec)
        def _():
          pltpu.make_async_copy(tbl.at[src_v.at[k]], rows_v[b],
                                sem[b]).wait()
          pltpu.sync_copy(rows_v[b], acc.at[dst_v.at[k]], add=True)
          nk = k + PIPE

          @pl.when(nk < k_per_tec)
          def _():
            gather(nk, b)
      return carry

    lax.fori_loop(0, -(-k_per_tec // PIPE), steps, 0)
    plsc.subcore_barrier()

    def acc_to_hbm(nf, tail):
      for j in range(nf):
        o = r0 + j * R
        pltpu.sync_copy(acc.at[pl.ds(o, R)], stage)
        pltpu.sync_copy(stage, out_hbm.at[c, pl.ds(o, R)])
      if tail:
        o = r0 + nf * R
        st = stage.at[pl.ds(0, tail)]
        pltpu.sync_copy(acc.at[pl.ds(o, tail)], st)
        pltpu.sync_copy(st, out_hbm.at[c, pl.ds(o, tail)])

    @pl.when(s < NS - 1)
    def _():
      acc_to_hbm(nf_rows, tail_rows)

    @pl.when(s == NS - 1)
    def _():
      acc_to_hbm(nf_last, tail_last)

  return pl.kernel(
      body,
      out_type=jax.ShapeDtypeStruct((NC, n, dh), jnp.float32),
      mesh=mesh,
      compiler_params=pltpu.CompilerParams(use_tc_tiling_on_sc=False),
      scratch_types=[
          pltpu.VMEM((k_per_tec, CHUNK), jnp.int32),
          pltpu.VMEM((k_per_tec, CHUNK), jnp.int32),
          [pltpu.VMEM((CHUNK, dh), jnp.float32)] * PIPE,
          pltpu.VMEM_SHARED((n, dh), jnp.float32),
          pltpu.VMEM_SHARED((acc_rows, dh), jnp.float32),
          [pltpu.SemaphoreType.DMA] * PIPE,
      ],
  )


def _make_deg(n, k_per_tec):
  rows, last = _node_split(n)
  acc_rows = NS * rows + 16
  kh = k_per_tec // 2
  mesh = plsc.VectorSubcoreMesh(core_axis_name="c", subcore_axis_name="s")

  def body(dst_hbm, out0_hbm, out1_hbm, dst_v, ones_v, stage_v, acc):
    c = lax.axis_index("c")
    s = lax.axis_index("s")
    r0 = s * rows

    def zfill(i, carry):
      stage_v[pl.ds(i * 16, 16)] = jnp.zeros((16,), jnp.float32)
      return carry

    lax.fori_loop(0, rows // 16, zfill, 0)

    @pl.when(s < NS - 1)
    def _():
      pltpu.sync_copy(stage_v.at[pl.ds(0, rows)], acc.at[pl.ds(r0, rows)])

    @pl.when(s == NS - 1)
    def _():
      pltpu.sync_copy(stage_v.at[pl.ds(0, last)], acc.at[pl.ds(r0, last)])

    for i in range(CHUNK // 16):
      ones_v[pl.ds(i * 16, 16)] = jnp.ones((16,), jnp.float32)
    pltpu.sync_copy(dst_hbm.at[s, pl.ds(c * kh, kh)], dst_v)
    plsc.subcore_barrier()

    def step(k, carry):
      pltpu.sync_copy(ones_v, acc.at[dst_v.at[k]], add=True)
      return carry

    lax.fori_loop(0, kh, step, 0)
    plsc.subcore_barrier()

    def acc_to_hbm(sz):
      pltpu.sync_copy(acc.at[pl.ds(r0, sz)], stage_v.at[pl.ds(0, sz)])

      @pl.when(c == 0)
      def _():
        pltpu.sync_copy(stage_v.at[pl.ds(0, sz)], out0_hbm.at[pl.ds(r0, sz)])

      @pl.when(c == 1)
      def _():
        pltpu.sync_copy(stage_v.at[pl.ds(0, sz)], out1_hbm.at[pl.ds(r0, sz)])

    @pl.when(s < NS - 1)
    def _():
      acc_to_hbm(rows)

    @pl.when(s == NS - 1)
    def _():
      acc_to_hbm(last)

  return pl.kernel(
      body,
      out_type=[jax.ShapeDtypeStruct((n,), jnp.float32),
                jax.ShapeDtypeStruct((n,), jnp.float32)],
      mesh=mesh,
      compiler_params=pltpu.CompilerParams(use_tc_tiling_on_sc=False),
      scratch_types=[
          pltpu.VMEM((kh, CHUNK), jnp.int32),
          pltpu.VMEM((CHUNK,), jnp.float32),
          pltpu.VMEM((rows,), jnp.float32),
          pltpu.VMEM_SHARED((acc_rows,), jnp.float32),
      ],
  )


def _elu(h):
  return jnp.where(h > 0, h, jnp.exp(jnp.minimum(h, 0.0)) - 1.0)


def _mm1_body(d0_ref, d1_ref, x_ref, w_ref, hp_ref, dinv_ref):
  i = pl.program_id(0)
  deg = d0_ref[i] + d1_ref[i] + 1.0
  dinv = lax.rsqrt(deg)
  dinv_ref[i] = dinv
  t = dinv[:, None] * jnp.dot(x_ref[...], w_ref[...],
                              preferred_element_type=jnp.float32)
  dh = t.shape[1] // 2
  hp_ref[0] = t[:, :dh]
  hp_ref[1] = t[:, dh:]


def _epimm_body(p_ref, dinv_ref, b_ref, w_ref, o_ref):
  dinv = dinv_ref[pl.program_id(0)][:, None]
  h = _elu(jnp.concatenate([p_ref[0], p_ref[1]], axis=1) * dinv + b_ref[...])
  t = dinv * jnp.dot(h, w_ref[...], preferred_element_type=jnp.float32)
  dh = t.shape[1] // 2
  o_ref[0] = t[:, :dh]
  o_ref[1] = t[:, dh:]


def _epi_body(p_ref, dinv_ref, b_ref, o_ref):
  h = jnp.concatenate([p_ref[0], p_ref[1]], axis=1)
  d_out = o_ref.shape[1]
  o_ref[...] = _elu(h * dinv_ref[pl.program_id(0)][:, None] + b_ref[...])[:, :d_out]


def kernel(x, edge_index, W1, b1, W2, b2, W3, b3):
  n, d_in = x.shape
  e = edge_index.shape[1]
  d_hid = W1.shape[1]
  d_out = W3.shape[1]
  dh = d_hid // 2
  d3 = 32
  dh3 = d3 // 2

  k_per_tec = -(-(-(-e // (NS * CHUNK))) // 2) * 2
  e_pad = NS * CHUNK * k_per_tec
  src = edge_index[0].astype(jnp.int32)
  dst = edge_index[1].astype(jnp.int32)
  pad = e_pad - e
  pad_dst = n + jnp.arange(pad, dtype=jnp.int32) % 128
  src2d = jnp.concatenate(
      [src, jnp.zeros((pad,), jnp.int32)]).reshape(NS, k_per_tec, CHUNK)
  dst2d = jnp.concatenate([dst, pad_dst]).reshape(NS, k_per_tec, CHUNK)

  W3p = jnp.pad(W3, ((0, 0), (0, d3 - d_out)))
  b3p = jnp.pad(b3, (0, d3 - d_out)).reshape(1, d3)

  agg_h = _make_agg(n, dh, k_per_tec)
  agg_3 = _make_agg(n, dh3, k_per_tec)
  deg_k = _make_deg(n, k_per_tec)

  bn = n
  grid = 1
  row_blk = lambda d: pl.BlockSpec((bn, d), lambda i: (i, 0))
  v_blk = pl.BlockSpec((grid, bn), lambda i: (0, 0))
  p_blk = lambda d: pl.BlockSpec((NC, bn, d), lambda i: (0, i, 0))
  full = lambda *shape: pl.BlockSpec(shape, lambda i: (0,) * len(shape))

  deg0, deg1 = deg_k(dst2d)

  hp1, dinv = pl.pallas_call(
      _mm1_body,
      grid=(grid,),
      in_specs=[v_blk, v_blk, row_blk(d_in), full(d_in, d_hid)],
      out_specs=[p_blk(dh), v_blk],
      out_shape=[jax.ShapeDtypeStruct((NC, n, dh), jnp.float32),
                 jax.ShapeDtypeStruct((grid, bn), jnp.float32)],
  )(deg0.reshape(grid, bn), deg1.reshape(grid, bn), x, W1)

  p1 = agg_h(src2d, dst2d, hp1)

  hp2 = pl.pallas_call(
      _epimm_body,
      grid=(grid,),
      in_specs=[p_blk(dh), v_blk, full(1, d_hid), full(d_hid, d_hid)],
      out_specs=p_blk(dh),
      out_shape=jax.ShapeDtypeStruct((NC, n, dh), jnp.float32),
  )(p1, dinv, b1.reshape(1, d_hid), W2)

  p2 = agg_h(src2d, dst2d, hp2)

  hp3 = pl.pallas_call(
      _epimm_body,
      grid=(grid,),
      in_specs=[p_blk(dh), v_blk, full(1, d_hid), full(d_hid, d3)],
      out_specs=p_blk(dh3),
      out_shape=jax.ShapeDtypeStruct((NC, n, dh3), jnp.float32),
  )(p2, dinv, b2.reshape(1, d_hid), W3p)

  p3 = agg_3(src2d, dst2d, hp3)

  out = pl.pallas_call(
      _epi_body,
      grid=(grid,),
      in_specs=[p_blk(dh3), v_blk, full(1, d3)],
      out_specs=row_blk(d_out),
      out_shape=jax.ShapeDtypeStruct((n, d_out), jnp.float32),
  )(p3, dinv, b3p)

  return out


# speedup vs baseline: 2.3531x; 1.0266x over previous
import jax
import jax.numpy as jnp
from jax import lax
from jax.experimental import pallas as pl
from jax.experimental.pallas import tpu as pltpu
from jax.experimental.pallas import tpu_sc as plsc

NC = 2
NS = 16
CHUNK = 128
PIPE = 4


def _node_split(n):
  rows = ((n + NS - 1) // NS + 15) // 16 * 16
  last = n - (NS - 1) * rows
  assert last > 0 and last % 16 == 0 and rows % 16 == 0
  return rows, last


def _make_agg(n, dh, k_per_tec):
  rows, last = _node_split(n)
  acc_rows = NS * rows + 16
  mesh = plsc.VectorSubcoreMesh(core_axis_name="c", subcore_axis_name="s")

  R = CHUNK
  nf_rows, tail_rows = rows // R, rows % R
  nf_last, tail_last = last // R, last % R

  def body(src_hbm, dst_hbm, hp_hbm, out_hbm, src_v, dst_v, rows_v,
           tbl, acc, sem):
    stage = rows_v[0]
    c = lax.axis_index("c")
    s = lax.axis_index("s")
    r0 = s * rows

    def load_tbl(nf, tail):
      hops = [(r0 + j * R, R) for j in range(nf)]
      if tail:
        hops.append((r0 + nf * R, tail))
      for j, (o, sz) in enumerate(hops[:PIPE]):
        pltpu.async_copy(hp_hbm.at[c, pl.ds(o, sz)],
                         rows_v[j].at[pl.ds(0, sz)], sem[j])
      for j, (o, sz) in enumerate(hops):
        st = rows_v[j % PIPE].at[pl.ds(0, sz)]
        pltpu.make_async_copy(hp_hbm.at[c, pl.ds(o, sz)], st,
                              sem[j % PIPE]).wait()
        pltpu.sync_copy(st, tbl.at[pl.ds(o, sz)])
        pltpu.sync_copy(st, acc.at[pl.ds(o, sz)])
        nj = j + PIPE
        if nj < len(hops):
          no, nsz = hops[nj]
          pltpu.async_copy(hp_hbm.at[c, pl.ds(no, nsz)],
                           rows_v[j % PIPE].at[pl.ds(0, nsz)], sem[j % PIPE])

    @pl.when(s < NS - 1)
    def _():
      load_tbl(nf_rows, tail_rows)

    @pl.when(s == NS - 1)
    def _():
      load_tbl(nf_last, tail_last)

    pltpu.sync_copy(src_hbm.at[s], src_v)
    pltpu.sync_copy(dst_hbm.at[s], dst_v)
    plsc.subcore_barrier()

    def gather(k, b):
      return pltpu.async_copy(tbl.at[src_v.at[k]], rows_v[b], sem[b])

    for b in range(PIPE):
      gather(b, b)

    def steps(i, carry):
      for b in range(PIPE):
        k = i * PIPE + b

        @pl.when(k < k_per_tec)
        def _():
          pltpu.make_async_copy(tbl.at[src_v.at[k]], rows_v[b],
                                sem[b]).wait()
          pltpu.sync_copy(rows_v[b], acc.at[dst_v.at[k]], add=True)
          nk = k + PIPE

          @pl.when(nk < k_per_tec)
          def _():
            gather(nk, b)
      return carry

    lax.fori_loop(0, -(-k_per_tec // PIPE), steps, 0)
    plsc.subcore_barrier()

    def acc_to_hbm(nf, tail):
      for j in range(nf):
        o = r0 + j * R
        pltpu.sync_copy(acc.at[pl.ds(o, R)], stage)
        pltpu.sync_copy(stage, out_hbm.at[c, pl.ds(o, R)])
      if tail:
        o = r0 + nf * R
        st = stage.at[pl.ds(0, tail)]
        pltpu.sync_copy(acc.at[pl.ds(o, tail)], st)
        pltpu.sync_copy(st, out_hbm.at[c, pl.ds(o, tail)])

    @pl.when(s < NS - 1)
    def _():
      acc_to_hbm(nf_rows, tail_rows)

    @pl.when(s == NS - 1)
    def _():
      acc_to_hbm(nf_last, tail_last)

  return pl.kernel(
      body,
      out_type=jax.ShapeDtypeStruct((NC, n, dh), jnp.float32),
      mesh=mesh,
      compiler_params=pltpu.CompilerParams(use_tc_tiling_on_sc=False),
      scratch_types=[
          pltpu.VMEM((k_per_tec, CHUNK), jnp.int32),
          pltpu.VMEM((k_per_tec, CHUNK), jnp.int32),
          [pltpu.VMEM((CHUNK, dh), jnp.float32)] * PIPE,
          pltpu.VMEM_SHARED((n, dh), jnp.float32),
          pltpu.VMEM_SHARED((acc_rows, dh), jnp.float32),
          [pltpu.SemaphoreType.DMA] * PIPE,
      ],
  )


def _make_deg(n, k_per_tec):
  rows, last = _node_split(n)
  acc_rows = NS * rows + 16
  kh = k_per_tec // 2
  mesh = plsc.VectorSubcoreMesh(core_axis_name="c", subcore_axis_name="s")

  def body(dst_hbm, out0_hbm, out1_hbm, dst_v, ones_v, stage_v, acc):
    c = lax.axis_index("c")
    s = lax.axis_index("s")
    r0 = s * rows

    def zfill(i, carry):
      stage_v[pl.ds(i * 16, 16)] = jnp.zeros((16,), jnp.float32)
      return carry

    lax.fori_loop(0, rows // 16, zfill, 0)

    @pl.when(s < NS - 1)
    def _():
      pltpu.sync_copy(stage_v.at[pl.ds(0, rows)], acc.at[pl.ds(r0, rows)])

    @pl.when(s == NS - 1)
    def _():
      pltpu.sync_copy(stage_v.at[pl.ds(0, last)], acc.at[pl.ds(r0, last)])

    for i in range(CHUNK // 16):
      ones_v[pl.ds(i * 16, 16)] = jnp.ones((16,), jnp.float32)
    pltpu.sync_copy(dst_hbm.at[s, pl.ds(c * kh, kh)], dst_v)
    plsc.subcore_barrier()

    def step(k, carry):
      pltpu.sync_copy(ones_v, acc.at[dst_v.at[k]], add=True)
      return carry

    lax.fori_loop(0, kh, step, 0)
    plsc.subcore_barrier()

    def acc_to_hbm(sz):
      pltpu.sync_copy(acc.at[pl.ds(r0, sz)], stage_v.at[pl.ds(0, sz)])

      @pl.when(c == 0)
      def _():
        pltpu.sync_copy(stage_v.at[pl.ds(0, sz)], out0_hbm.at[pl.ds(r0, sz)])

      @pl.when(c == 1)
      def _():
        pltpu.sync_copy(stage_v.at[pl.ds(0, sz)], out1_hbm.at[pl.ds(r0, sz)])

    @pl.when(s < NS - 1)
    def _():
      acc_to_hbm(rows)

    @pl.when(s == NS - 1)
    def _():
      acc_to_hbm(last)

  return pl.kernel(
      body,
      out_type=[jax.ShapeDtypeStruct((n,), jnp.float32),
                jax.ShapeDtypeStruct((n,), jnp.float32)],
      mesh=mesh,
      compiler_params=pltpu.CompilerParams(use_tc_tiling_on_sc=False),
      scratch_types=[
          pltpu.VMEM((kh, CHUNK), jnp.int32),
          pltpu.VMEM((CHUNK,), jnp.float32),
          pltpu.VMEM((rows,), jnp.float32),
          pltpu.VMEM_SHARED((acc_rows,), jnp.float32),
      ],
  )


def _elu(h):
  return jnp.where(h > 0, h, jnp.exp(jnp.minimum(h, 0.0)) - 1.0)


def _mm1_body(d0_ref, d1_ref, x_ref, w_ref, hp_ref, dinv_ref):
  i = pl.program_id(0)
  deg = d0_ref[i] + d1_ref[i] + 1.0
  dinv = lax.rsqrt(deg)
  dinv_ref[i] = dinv
  t = dinv[:, None] * jnp.dot(x_ref[...], w_ref[...],
                              preferred_element_type=jnp.float32)
  dh = t.shape[1] // 2
  hp_ref[0] = t[:, :dh]
  hp_ref[1] = t[:, dh:]


def _epimm_body(p_ref, dinv_ref, b_ref, w_ref, o_ref):
  dinv = dinv_ref[pl.program_id(0)][:, None]
  h = _elu(jnp.concatenate([p_ref[0], p_ref[1]], axis=1) * dinv + b_ref[...])
  t = dinv * jnp.dot(h, w_ref[...], preferred_element_type=jnp.float32)
  dh = t.shape[1] // 2
  o_ref[0] = t[:, :dh]
  o_ref[1] = t[:, dh:]


def _epi_body(p_ref, dinv_ref, b_ref, o_ref):
  h = jnp.concatenate([p_ref[0], p_ref[1]], axis=1)
  d_out = o_ref.shape[1]
  o_ref[...] = _elu(h * dinv_ref[pl.program_id(0)][:, None] + b_ref[...])[:, :d_out]


def kernel(x, edge_index, W1, b1, W2, b2, W3, b3):
  n, d_in = x.shape
  e = edge_index.shape[1]
  d_hid = W1.shape[1]
  d_out = W3.shape[1]
  dh = d_hid // 2
  d3 = 16
  dh3 = d3 // 2

  k_per_tec = -(-(-(-e // (NS * CHUNK))) // 2) * 2
  e_pad = NS * CHUNK * k_per_tec
  src = edge_index[0].astype(jnp.int32)
  dst = edge_index[1].astype(jnp.int32)
  pad = e_pad - e
  pad_dst = n + jnp.arange(pad, dtype=jnp.int32) % 128
  src2d = jnp.concatenate(
      [src, jnp.zeros((pad,), jnp.int32)]).reshape(NS, k_per_tec, CHUNK)
  dst2d = jnp.concatenate([dst, pad_dst]).reshape(NS, k_per_tec, CHUNK)

  W3p = jnp.pad(W3, ((0, 0), (0, d3 - d_out)))
  b3p = jnp.pad(b3, (0, d3 - d_out)).reshape(1, d3)

  agg_h = _make_agg(n, dh, k_per_tec)
  agg_3 = _make_agg(n, dh3, k_per_tec)
  deg_k = _make_deg(n, k_per_tec)

  bn = n
  grid = 1
  row_blk = lambda d: pl.BlockSpec((bn, d), lambda i: (i, 0))
  v_blk = pl.BlockSpec((grid, bn), lambda i: (0, 0))
  p_blk = lambda d: pl.BlockSpec((NC, bn, d), lambda i: (0, i, 0))
  full = lambda *shape: pl.BlockSpec(shape, lambda i: (0,) * len(shape))

  deg0, deg1 = deg_k(dst2d)

  hp1, dinv = pl.pallas_call(
      _mm1_body,
      grid=(grid,),
      in_specs=[v_blk, v_blk, row_blk(d_in), full(d_in, d_hid)],
      out_specs=[p_blk(dh), v_blk],
      out_shape=[jax.ShapeDtypeStruct((NC, n, dh), jnp.float32),
                 jax.ShapeDtypeStruct((grid, bn), jnp.float32)],
  )(deg0.reshape(grid, bn), deg1.reshape(grid, bn), x, W1)

  p1 = agg_h(src2d, dst2d, hp1)

  hp2 = pl.pallas_call(
      _epimm_body,
      grid=(grid,),
      in_specs=[p_blk(dh), v_blk, full(1, d_hid), full(d_hid, d_hid)],
      out_specs=p_blk(dh),
      out_shape=jax.ShapeDtypeStruct((NC, n, dh), jnp.float32),
  )(p1, dinv, b1.reshape(1, d_hid), W2)

  p2 = agg_h(src2d, dst2d, hp2)

  hp3 = pl.pallas_call(
      _epimm_body,
      grid=(grid,),
      in_specs=[p_blk(dh), v_blk, full(1, d_hid), full(d_hid, d3)],
      out_specs=p_blk(dh3),
      out_shape=jax.ShapeDtypeStruct((NC, n, dh3), jnp.float32),
  )(p2, dinv, b2.reshape(1, d_hid), W3p)

  p3 = agg_3(src2d, dst2d, hp3)

  out = pl.pallas_call(
      _epi_body,
      grid=(grid,),
      in_specs=[p_blk(dh3), v_blk, full(1, d3)],
      out_specs=row_blk(d_out),
      out_shape=jax.ShapeDtypeStruct((n, d_out), jnp.float32),
  )(p3, dinv, b3p)

  return out
